# baseline (speedup 1.0000x reference)
import functools

import jax
import jax.numpy as jnp
from jax import lax
from jax.experimental import pallas as pl
from jax.experimental.pallas import tpu as pltpu
from jax.experimental.pallas import tpu_sc as plsc

_NUM_CORES = 2
_NUM_SUBCORES = 16
_LANES = 16
_NBUF = 2


@functools.lru_cache(maxsize=None)
def _build(R, T, V, D):
    nw = _NUM_CORES * _NUM_SUBCORES
    rows_per_w = R // nw
    assert R % nw == 0 and rows_per_w % _NBUF == 0
    mesh = plsc.VectorSubcoreMesh(core_axis_name="c", subcore_axis_name="s")

    @functools.partial(
        pl.kernel,
        mesh=mesh,
        out_type=jax.ShapeDtypeStruct((R, T, D), jnp.float32),
        scratch_types=(
            [pltpu.VMEM((T,), jnp.int32) for _ in range(_NBUF)]
            + [pltpu.VMEM((T, 2 * D), jnp.float32) for _ in range(_NBUF)]
            + [pltpu.VMEM((T, D), jnp.float32) for _ in range(_NBUF)]
            + [pltpu.SemaphoreType.DMA for _ in range(2 * _NBUF)]
        ),
        compiler_params=pltpu.CompilerParams(use_tc_tiling_on_sc=True),
    )
    def emb_kernel(idx_hbm, table_hbm, out_hbm,
                   idx0, idx1, g0, g1, o0, o1, gs0, gs1, ss0, ss1):
        idxv = (idx0, idx1)
        gbuf = (g0, g1)
        obuf = (o0, o1)
        gsem = (gs0, gs1)
        ssem = (ss0, ss1)
        wid = lax.axis_index("s") * _NUM_CORES + lax.axis_index("c")
        base = wid * rows_per_w

        for b in range(_NBUF):
            pltpu.sync_copy(idx_hbm.at[pl.ds((base + b) * T, T)], idxv[b])
            pltpu.async_copy(table_hbm.at[idxv[b]], gbuf[b], gsem[b])

        def outer(o, _):
            for b in range(_NBUF):
                g = o * _NBUF + b
                row = base + g
                pltpu.make_async_copy(table_hbm.at[idxv[b]], gbuf[b],
                                      gsem[b]).wait()

                @pl.when(g + _NBUF < rows_per_w)
                def _():
                    pltpu.sync_copy(
                        idx_hbm.at[pl.ds((row + _NBUF) * T, T)], idxv[b])

                @pl.when(g >= _NBUF)
                def _():
                    pltpu.make_async_copy(
                        obuf[b], out_hbm.at[row - _NBUF], ssem[b]).wait()

                def sbody(r, _):
                    for j in range(D // _LANES):
                        sl = pl.ds(j * _LANES, _LANES)
                        obuf[b][r, sl] = gbuf[b][r, sl] * 8.0
                    return 0

                lax.fori_loop(0, T, sbody, 0, unroll=4)

                @pl.when(g + _NBUF < rows_per_w)
                def _():
                    pltpu.async_copy(table_hbm.at[idxv[b]], gbuf[b], gsem[b])

                pltpu.async_copy(obuf[b], out_hbm.at[row], ssem[b])
            return 0

        lax.fori_loop(0, rows_per_w // _NBUF, outer, 0)

        for b in range(_NBUF):
            row = base + rows_per_w - _NBUF + b
            pltpu.make_async_copy(obuf[b], out_hbm.at[row], ssem[b]).wait()

    return emb_kernel


def kernel(tokens, weight):
    dim1, dim2 = tokens.shape
    V, D = weight.shape
    idx = tokens.reshape(-1).astype(jnp.int32)
    wpad = jnp.pad(weight, ((0, 0), (0, D)))
    return _build(dim1, dim2, V, D)(idx, wpad)


# speedup vs baseline: 1.1399x; 1.1399x over previous
import functools

import jax
import jax.numpy as jnp
from jax import lax
from jax.experimental import pallas as pl
from jax.experimental.pallas import tpu as pltpu
from jax.experimental.pallas import tpu_sc as plsc

_NUM_CORES = 2
_NUM_SUBCORES = 16
_LANES = 16
_NBUF = 2


@functools.lru_cache(maxsize=None)
def _build(B, V, D, chunk):
    nw = _NUM_CORES * _NUM_SUBCORES
    per_w = B // nw
    nchunks = per_w // chunk
    assert per_w % chunk == 0 and nchunks % _NBUF == 0
    mesh = plsc.VectorSubcoreMesh(core_axis_name="c", subcore_axis_name="s")

    @functools.partial(
        pl.kernel,
        mesh=mesh,
        out_type=jax.ShapeDtypeStruct((B, D), jnp.float32),
        scratch_types=(
            [pltpu.VMEM((chunk,), jnp.int32) for _ in range(_NBUF)]
            + [pltpu.VMEM((chunk, D), jnp.float32) for _ in range(_NBUF)]
            + [pltpu.SemaphoreType.DMA for _ in range(2 * _NBUF)]
        ),
        compiler_params=pltpu.CompilerParams(use_tc_tiling_on_sc=False),
    )
    def emb_kernel(idx_hbm, table_hbm, out_hbm,
                   idx0, idx1, rows0, rows1, g0, g1, s0, s1):
        idxv = (idx0, idx1)
        rows = (rows0, rows1)
        gsem = (g0, g1)
        ssem = (s0, s1)
        wid = lax.axis_index("s") * _NUM_CORES + lax.axis_index("c")
        base = wid * per_w

        for b in range(_NBUF):
            pltpu.sync_copy(idx_hbm.at[pl.ds(base + b * chunk, chunk)],
                            idxv[b])
            pltpu.async_copy(table_hbm.at[idxv[b]], rows[b], gsem[b])

        def outer(o, _):
            for b in range(_NBUF):
                g = o * _NBUF + b
                pltpu.make_async_copy(table_hbm.at[idxv[b]], rows[b],
                                      gsem[b]).wait()

                def sbody(r, _):
                    for j in range(D // _LANES):
                        sl = pl.ds(j * _LANES, _LANES)
                        rows[b][r, sl] = rows[b][r, sl] * 8.0
                    return 0

                lax.fori_loop(0, chunk, sbody, 0, unroll=4)

                dst = out_hbm.at[pl.ds(base + g * chunk, chunk)]
                pltpu.async_copy(rows[b], dst, ssem[b])

                @pl.when(g + _NBUF < nchunks)
                def _():
                    pltpu.sync_copy(
                        idx_hbm.at[pl.ds(base + (g + _NBUF) * chunk, chunk)],
                        idxv[b])
                    pltpu.make_async_copy(rows[b], dst, ssem[b]).wait()
                    pltpu.async_copy(table_hbm.at[idxv[b]], rows[b], gsem[b])
            return 0

        lax.fori_loop(0, nchunks // _NBUF, outer, 0)

        for b in range(_NBUF):
            g = nchunks - _NBUF + b
            dst = out_hbm.at[pl.ds(base + g * chunk, chunk)]
            pltpu.make_async_copy(rows[b], dst, ssem[b]).wait()

    return emb_kernel


def kernel(tokens, weight):
    dim1, dim2 = tokens.shape
    V, D = weight.shape
    B = dim1 * dim2
    idx = tokens.T.reshape(-1).astype(jnp.int32)
    out = _build(B, V, D, 512)(idx, weight)
    return out.reshape(dim2, dim1, D).transpose(1, 0, 2)
